# baseline (speedup 1.0000x reference)
import functools

import jax
import jax.numpy as jnp
from jax import lax
from jax.experimental import pallas as pl
from jax.experimental.pallas import tpu as pltpu
from jax.experimental.pallas import tpu_sc as plsc

N = 200704
NG = N // 4
NC = 2
NS = 16
NW = NC * NS
B_PER_W = N // NW
CHUNK = 128
CHUNKS = B_PER_W // CHUNK
L = 16
GROUPS = CHUNK // L

_mesh = plsc.VectorSubcoreMesh(core_axis_name="c", subcore_axis_name="s")


@functools.partial(
    pl.kernel,
    mesh=_mesh,
    compiler_params=pltpu.CompilerParams(
        use_tc_tiling_on_sc=False, needs_layout_passes=False
    ),
    out_type=jax.ShapeDtypeStruct((2 * N,), jnp.float32),
    scratch_types=[
        pltpu.VMEM((CHUNKS, CHUNK), jnp.int32),
        pltpu.VMEM((CHUNK,), jnp.int32),
        pltpu.VMEM((CHUNK, 8), jnp.float32),
        pltpu.VMEM((2 * CHUNK,), jnp.float32),
        pltpu.SemaphoreType.DMA,
    ],
)
def _gather_kernel(table_hbm, idx_hbm, out_hbm, idx_v, g_v, rows_v, out_v, sem):
    wid = lax.axis_index("s") * NC + lax.axis_index("c")
    base = wid * B_PER_W
    pltpu.sync_copy(idx_hbm.at[wid], idx_v)

    @pl.loop(0, CHUNKS)
    def _chunk(j):
        for k in range(GROUPS):
            v = idx_v[j, pl.ds(L * k, L)]
            g_v[pl.ds(L * k, L)] = lax.shift_right_logical(v, 2)
        pltpu.async_copy(table_hbm.at[g_v], rows_v, sem).wait()
        for k in range(GROUPS):
            v = idx_v[j, pl.ds(L * k, L)]
            off2 = lax.shift_left(jnp.bitwise_and(v, 3), 1)
            row = lax.iota(jnp.int32, L) + (L * k)
            x = plsc.load_gather(rows_v, [row, off2])
            y = plsc.load_gather(rows_v, [row, off2 + 1])
            pos = lax.shift_left(row, 1)
            plsc.store_scatter(out_v, [pos], x)
            plsc.store_scatter(out_v, [pos + 1], y)
        pltpu.sync_copy(out_v, out_hbm.at[pl.ds(2 * (base + j * CHUNK), 2 * CHUNK)])


def kernel(positions, perm):
    table = positions.reshape(NG, 8)
    idx = perm.astype(jnp.int32).reshape(NW, CHUNKS, CHUNK)
    return _gather_kernel(table, idx).reshape(N, 2)


# speedup vs baseline: 1.1086x; 1.1086x over previous
import functools

import jax
import jax.numpy as jnp
from jax import lax
from jax.experimental import pallas as pl
from jax.experimental.pallas import tpu as pltpu
from jax.experimental.pallas import tpu_sc as plsc

N = 200704
NG = N // 4
NC = 2
NS = 16
NW = NC * NS
B_PER_W = N // NW
CHUNK = 128
CHUNKS = B_PER_W // CHUNK
L = 16
GROUPS = CHUNK // L

_mesh = plsc.VectorSubcoreMesh(core_axis_name="c", subcore_axis_name="s")


@functools.partial(
    pl.kernel,
    mesh=_mesh,
    compiler_params=pltpu.CompilerParams(
        use_tc_tiling_on_sc=False, needs_layout_passes=False
    ),
    out_type=jax.ShapeDtypeStruct((2 * N,), jnp.float32),
    scratch_types=[
        pltpu.VMEM((CHUNKS, CHUNK), jnp.int32),
        pltpu.VMEM((CHUNKS, CHUNK), jnp.int32),
        pltpu.VMEM((B_PER_W, 8), jnp.float32),
        pltpu.VMEM((2 * B_PER_W,), jnp.float32),
        pltpu.SemaphoreType.DMA,
    ],
)
def _gather_kernel(table_hbm, idx_hbm, out_hbm, idx_v, g_v, rows_v, out_v, sem):
    wid = lax.axis_index("s") * NC + lax.axis_index("c")
    base = wid * B_PER_W
    pltpu.sync_copy(idx_hbm.at[wid], idx_v)

    @pl.loop(0, CHUNKS)
    def _fire(j):
        for k in range(GROUPS):
            v = idx_v[j, pl.ds(L * k, L)]
            g_v[j, pl.ds(L * k, L)] = lax.shift_right_logical(v, 2)
        pltpu.async_copy(
            table_hbm.at[g_v.at[j]], rows_v.at[pl.ds(j * CHUNK, CHUNK)], sem
        )

    pltpu.make_async_copy(table_hbm.at[pl.ds(0, B_PER_W)], rows_v, sem).wait()

    @pl.loop(0, CHUNKS)
    def _extract(j):
        for k in range(GROUPS):
            v = idx_v[j, pl.ds(L * k, L)]
            off2 = lax.shift_left(jnp.bitwise_and(v, 3), 1)
            row = lax.iota(jnp.int32, L) + (j * CHUNK + L * k)
            x = plsc.load_gather(rows_v, [row, off2])
            y = plsc.load_gather(rows_v, [row, off2 + 1])
            pos = lax.shift_left(row, 1)
            plsc.store_scatter(out_v, [pos], x)
            plsc.store_scatter(out_v, [pos + 1], y)

    pltpu.sync_copy(out_v, out_hbm.at[pl.ds(2 * base, 2 * B_PER_W)])


def kernel(positions, perm):
    table = positions.reshape(NG, 8)
    idx = perm.astype(jnp.int32).reshape(NW, CHUNKS, CHUNK)
    return _gather_kernel(table, idx).reshape(N, 2)


# speedup vs baseline: 1.2711x; 1.1466x over previous
import functools

import jax
import jax.numpy as jnp
from jax import lax
from jax.experimental import pallas as pl
from jax.experimental.pallas import tpu as pltpu
from jax.experimental.pallas import tpu_sc as plsc

N = 200704
NG = N // 4
NC = 2
NS = 16
NW = NC * NS
B_PER_W = N // NW
CHUNK = 128
CHUNKS = B_PER_W // CHUNK
L = 16
GROUPS = CHUNK // L

_mesh = plsc.VectorSubcoreMesh(core_axis_name="c", subcore_axis_name="s")


@functools.partial(
    pl.kernel,
    mesh=_mesh,
    compiler_params=pltpu.CompilerParams(
        use_tc_tiling_on_sc=False, needs_layout_passes=False
    ),
    out_type=jax.ShapeDtypeStruct((N, 2), jnp.float32),
    scratch_types=[
        pltpu.VMEM((CHUNKS, CHUNK), jnp.int32),
        pltpu.VMEM((CHUNKS, CHUNK), jnp.int32),
        pltpu.VMEM((B_PER_W, 8), jnp.float32),
        pltpu.VMEM((B_PER_W, 2), jnp.float32),
        pltpu.SemaphoreType.DMA,
    ],
)
def _gather_kernel(table_hbm, idx_hbm, out_hbm, idx_v, g_v, rows_v, out_v, sem):
    wid = lax.axis_index("s") * NC + lax.axis_index("c")
    base = wid * B_PER_W
    pltpu.sync_copy(idx_hbm.at[pl.ds(wid * CHUNKS, CHUNKS)], idx_v)

    @pl.loop(0, CHUNKS)
    def _fire(j):
        for k in range(GROUPS):
            v = idx_v[j, pl.ds(L * k, L)]
            g_v[j, pl.ds(L * k, L)] = lax.shift_right_logical(v, 2)
        pltpu.async_copy(
            table_hbm.at[g_v.at[j]], rows_v.at[pl.ds(j * CHUNK, CHUNK)], sem
        )

    pltpu.make_async_copy(table_hbm.at[pl.ds(0, B_PER_W)], rows_v, sem).wait()

    czero = jnp.zeros((L,), jnp.int32)

    @pl.loop(0, CHUNKS)
    def _extract(j):
        for k in range(GROUPS):
            v = idx_v[j, pl.ds(L * k, L)]
            off2 = lax.shift_left(jnp.bitwise_and(v, 3), 1)
            row = lax.iota(jnp.int32, L) + (j * CHUNK + L * k)
            x = plsc.load_gather(rows_v, [row, off2])
            y = plsc.load_gather(rows_v, [row, off2 + 1])
            plsc.store_scatter(out_v, [row, czero], x)
            plsc.store_scatter(out_v, [row, czero + 1], y)

    pltpu.sync_copy(out_v, out_hbm.at[pl.ds(base, B_PER_W)])


def kernel(positions, perm):
    table = positions.reshape(NG, 8)
    idx = perm.astype(jnp.int32).reshape(N // CHUNK, CHUNK)
    return _gather_kernel(table, idx)
